# baseline (speedup 1.0000x reference)
import functools

import jax
import jax.numpy as jnp
from jax import lax
from jax.experimental import pallas as pl
from jax.experimental.pallas import tpu as pltpu
from jax.experimental.pallas import tpu_sc as plsc

_NC = 2
_NS = 16
_NW = _NC * _NS
_LANES = 16

N = 10000
E = 320000
HIDDEN = 128
EMB1 = 128
EMB2 = 64
L1 = 64
G = 64


def _tc1_body(e_raw_ref, w1root_ref, b1_ref, epair_ref, eroot_ref):
    e_raw = e_raw_ref[0:N, :]
    nrm2 = jnp.sum(e_raw * e_raw, axis=1, keepdims=True)
    scale = jnp.where(nrm2 > 1.0, lax.rsqrt(nrm2), 1.0)
    e = e_raw * scale
    epair_ref[0] = e[:, 0:_HC]
    epair_ref[1] = e[:, _HC:2 * _HC]
    eroot_ref[:] = (
        jax.lax.dot_general(e, w1root_ref[:], (((1,), (1,)), ((), ())),
                            preferred_element_type=jnp.float32)
        + b1_ref[:][None, :]
    )


def _tc1(e_raw_pad, W1_root, b1_rel):
    return pl.pallas_call(
        _tc1_body,
        out_shape=(
            jax.ShapeDtypeStruct((2, N, _HC), jnp.float32),
            jax.ShapeDtypeStruct((N, EMB1), jnp.float32),
        ),
    )(e_raw_pad, W1_root, b1_rel)


def _mm_halves(aggp_ref, w):
    def mm(a, wslice):
        return jax.lax.dot_general(a, wslice, (((1,), (1,)), ((), ())),
                                   preferred_element_type=jnp.float32)
    return (mm(aggp_ref[0:N, :], w[:, 0:_HC])
            + mm(aggp_ref[_NROW:_NROW + N, :], w[:, _HC:2 * _HC]))


def _tc3_body(aggp_ref, w1rel_ref, eroot_ref, h_ref, hpair_ref):
    h = _mm_halves(aggp_ref, w1rel_ref[:]) + eroot_ref[:]
    h = jnp.maximum(h, 0.0)
    h_ref[:] = h
    hpair_ref[0] = h[:, 0:_HC]
    hpair_ref[1] = h[:, _HC:2 * _HC]


def _tc3(agg1p, W1_rel, eroot):
    return pl.pallas_call(
        _tc3_body,
        out_shape=(
            jax.ShapeDtypeStruct((N, EMB1), jnp.float32),
            jax.ShapeDtypeStruct((2, N, _HC), jnp.float32),
        ),
    )(agg1p, W1_rel, eroot)


def _tc5_body(agg2_ref, h_ref, wmu_rel_ref, bmu_ref, wmu_root_ref,
              wstd_rel_ref, bstd_ref, wstd_root_ref, eps_ref, batch_ref,
              wc1_ref, bc1_ref, wc2_ref, bc2_ref, logstd_ref,
              z_ref, zmu_ref, zstd_ref, zn_ref, y_ref, wstd_out_ref):
    h = h_ref[:]

    def mm_t(a, w):
        return jax.lax.dot_general(a, w, (((1,), (1,)), ((), ())),
                                   preferred_element_type=jnp.float32)

    z_mu = jnp.tanh(_mm_halves(agg2_ref, wmu_rel_ref[:]) + bmu_ref[:][None, :]
                    + mm_t(h, wmu_root_ref[:]))
    z_ls = jnp.tanh(_mm_halves(agg2_ref, wstd_rel_ref[:]) + bstd_ref[:][None, :]
                    + mm_t(h, wstd_root_ref[:]))
    z_std = jnp.exp(z_ls)
    z = z_mu + z_std * eps_ref[:]
    zmu_ref[:] = z_mu
    zstd_ref[:] = z_std
    z_ref[:] = z
    zn2 = jnp.sum(z * z, axis=1, keepdims=True)
    rinv = 1.0 / jnp.maximum(jnp.sqrt(zn2), 1e-8)
    zn_ref[:] = z * rinv
    seg = lax.broadcasted_iota(jnp.int32, (G, N), 0)
    mask = (batch_ref[:][None, :] == seg).astype(jnp.float32)
    cnt = jnp.sum(mask, axis=1, keepdims=True)
    pooled = jax.lax.dot_general(mask, z_mu, (((1,), (0,)), ((), ())),
                                 preferred_element_type=jnp.float32)
    pooled = pooled / jnp.maximum(cnt, 1.0)
    y = jnp.maximum(mm_t(pooled, wc1_ref[:]) + bc1_ref[:][None, :], 0.0)
    y = mm_t(y, wc2_ref[:]) + bc2_ref[:][None, :]
    y = y - jnp.max(y, axis=1, keepdims=True)
    ey = jnp.exp(y)
    y_ref[:] = ey / jnp.sum(ey, axis=1, keepdims=True)
    wstd_out_ref[:] = jnp.exp(logstd_ref[:])


def _tc5(agg2, h, Wmu_rel, bmu_rel, Wmu_root, Wstd_rel, bstd_rel, Wstd_root,
         eps, batch, Wc1, bc1, Wc2, bc2, log_std):
    return pl.pallas_call(
        _tc5_body,
        out_shape=(
            jax.ShapeDtypeStruct((N, EMB2), jnp.float32),
            jax.ShapeDtypeStruct((N, EMB2), jnp.float32),
            jax.ShapeDtypeStruct((N, EMB2), jnp.float32),
            jax.ShapeDtypeStruct((N, EMB2), jnp.float32),
            jax.ShapeDtypeStruct((G, 2), jnp.float32),
            jax.ShapeDtypeStruct((1,), jnp.float32),
        ),
    )(agg2, h, Wmu_rel, bmu_rel, Wmu_root, Wstd_rel, bstd_rel, Wstd_root,
      eps, batch, Wc1, bc1, Wc2, bc2, log_std)


def _chunk_sizes(total, cap=128):
    out = []
    while total > 0:
        c = min(cap, total)
        out.append(c)
        total -= c
    return out


@functools.partial(jax.jit, static_argnames=("n_rows", "n_cols"))
def _sc_gather(table, idx, n_rows, n_cols):
    bpw = n_rows // _NW
    mesh = plsc.VectorSubcoreMesh(core_axis_name="c", subcore_axis_name="s")

    @functools.partial(
        pl.kernel, mesh=mesh,
        out_type=jax.ShapeDtypeStruct((n_rows, n_cols), jnp.float32),
        scratch_types=[
            pltpu.VMEM((bpw,), jnp.int32),
            pltpu.VMEM((bpw, n_cols), jnp.float32),
            pltpu.SemaphoreType.DMA,
        ],
    )
    def k(table_hbm, idx_hbm, out_hbm, idx_v, rows_v, sem):
        wid = lax.axis_index("s") * _NC + lax.axis_index("c")
        base = wid * bpw
        pltpu.sync_copy(idx_hbm.at[pl.ds(base, bpw)], idx_v)
        copies = []
        off = 0
        for cs in _chunk_sizes(bpw):
            copies.append(pltpu.async_copy(
                table_hbm.at[idx_v.at[pl.ds(off, cs)]],
                rows_v.at[pl.ds(off, cs)], sem))
            off += cs
        for c in copies:
            c.wait()
        pltpu.sync_copy(rows_v, out_hbm.at[pl.ds(base, bpw)])

    return k(table, idx)


def _gather_rows(table, idx):
    return table[idx]


_C = 128
_E_PAD = 327680
_NROW = 10240
_RPS = _NROW // _NS
_HC = 64
_NCH2 = _E_PAD // (_NS * _C)
_EPS = _NCH2 * _C
_EPW = _E_PAD // _NW
_DCH = _EPW // _C


@jax.jit
def _sc_segsum(table_pair, srcp, dstp, ewp):
    mesh = plsc.VectorSubcoreMesh(core_axis_name="c", subcore_axis_name="s")
    cvecs = _HC // _LANES

    @functools.partial(
        pl.kernel, mesh=mesh,
        out_type=jax.ShapeDtypeStruct((2 * _NROW, _HC), jnp.float32),
        compiler_params=pltpu.CompilerParams(needs_layout_passes=False,
                                             use_tc_tiling_on_sc=False),
        scratch_types=[
            pltpu.VMEM((_NCH2, _C), jnp.int32),
            pltpu.VMEM((_NCH2, _C), jnp.int32),
            pltpu.VMEM((_EPS,), jnp.float32),
            pltpu.VMEM((_C, _HC), jnp.float32),
            pltpu.VMEM((_C, _HC), jnp.float32),
            pltpu.VMEM_SHARED((_NROW, _HC), jnp.float32),
            pltpu.SemaphoreType.DMA,
            pltpu.SemaphoreType.DMA,
        ],
    )
    def k(x_hbm, src_hbm, dst_hbm, ew_hbm, out_hbm,
          src_v, dst_v, ew_v, msg0_v, msg1_v, agg_sh, sem0, sem1):
        cid = lax.axis_index("c")
        sid = lax.axis_index("s")
        zv = jnp.zeros((_LANES,), jnp.float32)

        def zrow(i, c2):
            for c in range(_HC // _LANES):
                msg0_v[i, pl.ds(c * _LANES, _LANES)] = zv
            return c2

        lax.fori_loop(0, _C, zrow, 0)
        for b in range(_RPS // _C):
            pltpu.sync_copy(msg0_v,
                            agg_sh.at[pl.ds(sid * _RPS + b * _C, _C)])
        pltpu.sync_copy(src_hbm.at[sid], src_v)
        pltpu.sync_copy(dst_hbm.at[sid], dst_v)
        pltpu.sync_copy(ew_hbm.at[sid], ew_v)
        plsc.subcore_barrier()
        xh = x_hbm.at[cid]

        def scale_and_scatter(j, msg_v):
            jbase = jnp.full((_LANES,), j * _C, jnp.int32)

            def srow(i4, c2):
                for u in range(4):
                    i = i4 * 4 + u
                    w = plsc.load_gather(ew_v, [jbase + i])
                    for c in range(cvecs):
                        msg_v[i, pl.ds(c * _LANES, _LANES)] = (
                            msg_v[i, pl.ds(c * _LANES, _LANES)] * w)
                return c2

            lax.fori_loop(0, _C // 4, srow, 0)
            pltpu.sync_copy(msg_v, agg_sh.at[dst_v.at[j]], add=True)

        pltpu.async_copy(xh.at[src_v.at[0]], msg0_v, sem0)

        def pair(jj, carry):
            j = jj * 2
            c1 = pltpu.async_copy(xh.at[src_v.at[j + 1]], msg1_v, sem1)
            pltpu.make_async_copy(xh.at[src_v.at[j]], msg0_v, sem0).wait()
            scale_and_scatter(j, msg0_v)
            pltpu.async_copy(xh.at[src_v.at[j + 2]], msg0_v, sem0)
            c1.wait()
            scale_and_scatter(j + 1, msg1_v)
            return carry

        lax.fori_loop(0, _NCH2 // 2 - 1, pair, 0)
        jl = _NCH2 - 2
        cl = pltpu.async_copy(xh.at[src_v.at[jl + 1]], msg1_v, sem1)
        pltpu.make_async_copy(xh.at[src_v.at[jl]], msg0_v, sem0).wait()
        scale_and_scatter(jl, msg0_v)
        cl.wait()
        scale_and_scatter(jl + 1, msg1_v)

        plsc.subcore_barrier()
        pltpu.sync_copy(agg_sh.at[pl.ds(sid * _RPS, _RPS)],
                        out_hbm.at[pl.ds(cid * _NROW + sid * _RPS, _RPS)])

    return k(table_pair, srcp, dstp, ewp)


def _segsum(xrows, ew, dst):
    return jax.ops.segment_sum(xrows * ew[:, None], dst, num_segments=N)


@jax.jit
def _sc_edge_dot(zn, srcp, dstp):
    mesh = plsc.VectorSubcoreMesh(core_axis_name="c", subcore_axis_name="s")
    cvecs = EMB2 // _LANES

    @functools.partial(
        pl.kernel, mesh=mesh,
        out_type=jax.ShapeDtypeStruct((E,), jnp.float32),
        compiler_params=pltpu.CompilerParams(needs_layout_passes=False,
                                             use_tc_tiling_on_sc=False),
        scratch_types=[
            pltpu.VMEM((_EPW,), jnp.int32),
            pltpu.VMEM((_EPW,), jnp.int32),
            pltpu.VMEM((_C, EMB2), jnp.float32),
            pltpu.VMEM((_C, EMB2), jnp.float32),
            pltpu.VMEM((_C, EMB2), jnp.float32),
            pltpu.VMEM((_C, EMB2), jnp.float32),
            pltpu.VMEM((_EPW,), jnp.float32),
            pltpu.SemaphoreType.DMA,
            pltpu.SemaphoreType.DMA,
        ],
    )
    def k(zn_hbm, src_hbm, dst_hbm, out_hbm, src_v, dst_v,
          a0_v, b0_v, a1_v, b1_v, o_v, sem0, sem1):
        cid = lax.axis_index("c")
        sid = lax.axis_index("s")
        wid = sid * _NC + cid
        pltpu.sync_copy(src_hbm.at[wid], src_v)
        pltpu.sync_copy(dst_hbm.at[wid], dst_v)
        lane = lax.broadcasted_iota(jnp.int32, (_LANES,), 0)
        last = lane == (_LANES - 1)

        def gather_ab(j, a_v, b_v, sem):
            pltpu.async_copy(zn_hbm.at[src_v.at[pl.ds(j * _C, _C)]], a_v, sem)
            pltpu.async_copy(zn_hbm.at[dst_v.at[pl.ds(j * _C, _C)]], b_v, sem)

        def wait_ab(j, a_v, b_v, sem):
            pltpu.make_async_copy(zn_hbm.at[src_v.at[pl.ds(j * _C, _C)]],
                                  a_v, sem).wait()
            pltpu.make_async_copy(zn_hbm.at[dst_v.at[pl.ds(j * _C, _C)]],
                                  b_v, sem).wait()

        def dots(j, a_v, b_v):
            jbase = jnp.full((_LANES,), j * _C, jnp.int32)

            def edge(i8, c2):
                for u in range(8):
                    i = i8 * 8 + u
                    s = (a_v[i, pl.ds(0, _LANES)] * b_v[i, pl.ds(0, _LANES)])
                    for c in range(1, cvecs):
                        s = s + (a_v[i, pl.ds(c * _LANES, _LANES)]
                                 * b_v[i, pl.ds(c * _LANES, _LANES)])
                    cs = plsc.cumsum(s)
                    plsc.store_scatter(o_v, [jbase + i], cs, mask=last)
                return c2

            lax.fori_loop(0, _C // 8, edge, 0)

        gather_ab(0, a0_v, b0_v, sem0)

        def pair(jj, carry):
            j = jj * 2
            gather_ab(j + 1, a1_v, b1_v, sem1)
            wait_ab(j, a0_v, b0_v, sem0)
            dots(j, a0_v, b0_v)
            gather_ab(j + 2, a0_v, b0_v, sem0)
            wait_ab(j + 1, a1_v, b1_v, sem1)
            dots(j + 1, a1_v, b1_v)
            return carry

        lax.fori_loop(0, _DCH // 2 - 1, pair, 0)
        jl = _DCH - 2
        gather_ab(jl + 1, a1_v, b1_v, sem1)
        wait_ab(jl, a0_v, b0_v, sem0)
        dots(jl, a0_v, b0_v)
        wait_ab(jl + 1, a1_v, b1_v, sem1)
        dots(jl + 1, a1_v, b1_v)
        tail = E - (_NW - 1) * _EPW

        @pl.when(wid < _NW - 1)
        def _():
            pltpu.sync_copy(o_v, out_hbm.at[pl.ds(wid * _EPW, _EPW)])

        @pl.when(wid == _NW - 1)
        def _():
            pltpu.sync_copy(o_v.at[pl.ds(0, tail)],
                            out_hbm.at[pl.ds((_NW - 1) * _EPW, tail)])

    return k(zn, srcp, dstp)


def kernel(x, edge_index, edge_weight, batch, emb_table, W1_rel, b1_rel,
           W1_root, Wmu_rel, bmu_rel, Wmu_root, Wstd_rel, bstd_rel, Wstd_root,
           Wc1, bc1, Wc2, bc2, log_std, eps):
    src = edge_index[0]
    dst = edge_index[1]

    n_pad = 10240
    x_pad = jnp.pad(x, (0, n_pad - N))
    e_raw_pad = _sc_gather(emb_table, x_pad, n_pad, HIDDEN)
    e_pair, eroot = _tc1(e_raw_pad, W1_root, b1_rel)

    pe = _E_PAD - E
    src_pad = jnp.pad(src, (0, pe))
    dst_pad = jnp.pad(dst, (0, pe))
    ew_pad = jnp.pad(edge_weight, (0, pe))
    srcp = src_pad.reshape(_NW, _EPW)
    dstp = dst_pad.reshape(_NW, _EPW)
    srcp16 = src_pad.reshape(_NS, _NCH2, _C)
    dstp16 = dst_pad.reshape(_NS, _NCH2, _C)
    ewp16 = ew_pad.reshape(_NS, _EPS)
    agg1p = _sc_segsum(e_pair, srcp16, dstp16, ewp16)
    h, h_pair = _tc3(agg1p, W1_rel, eroot)

    agg2p = _sc_segsum(h_pair, srcp16, dstp16, ewp16)
    z, z_mu, z_std, zn, y, w_std = _tc5(
        agg2p, h, Wmu_rel, bmu_rel, Wmu_root, Wstd_rel, bstd_rel, Wstd_root,
        eps, batch, Wc1, bc1, Wc2, bc2, log_std)

    w_mu = _sc_edge_dot(zn, srcp, dstp)
    return (y, w_mu, w_std, z, z_mu, z_std)


# speedup vs baseline: 1.8738x; 1.8738x over previous
import functools

import jax
import jax.numpy as jnp
from jax import lax
from jax.experimental import pallas as pl
from jax.experimental.pallas import tpu as pltpu
from jax.experimental.pallas import tpu_sc as plsc

_NC = 2
_NS = 16
_NW = _NC * _NS
_LANES = 16

N = 10000
E = 320000
HIDDEN = 128
EMB1 = 128
EMB2 = 64
L1 = 64
G = 64


def _tc1_body(e_raw_ref, w1root_ref, b1_ref, epair_ref, eroot_ref):
    e_raw = e_raw_ref[0:N, :]
    nrm2 = jnp.sum(e_raw * e_raw, axis=1, keepdims=True)
    scale = jnp.where(nrm2 > 1.0, lax.rsqrt(nrm2), 1.0)
    e = e_raw * scale
    epair_ref[0] = e[:, 0:_HC]
    epair_ref[1] = e[:, _HC:2 * _HC]
    eroot_ref[:] = (
        jax.lax.dot_general(e, w1root_ref[:], (((1,), (1,)), ((), ())),
                            preferred_element_type=jnp.float32)
        + b1_ref[:][None, :]
    )


def _tc1(e_raw_pad, W1_root, b1_rel):
    return pl.pallas_call(
        _tc1_body,
        out_shape=(
            jax.ShapeDtypeStruct((2, N, _HC), jnp.float32),
            jax.ShapeDtypeStruct((N, EMB1), jnp.float32),
        ),
    )(e_raw_pad, W1_root, b1_rel)


def _mm_halves(aggp_ref, w):
    def mm(a, wslice):
        return jax.lax.dot_general(a, wslice, (((1,), (1,)), ((), ())),
                                   preferred_element_type=jnp.float32)
    return (mm(aggp_ref[0:N, :], w[:, 0:_HC])
            + mm(aggp_ref[_NROW:_NROW + N, :], w[:, _HC:2 * _HC]))


def _tc3_body(aggp_ref, w1rel_ref, eroot_ref, h_ref, hpair_ref):
    h = _mm_halves(aggp_ref, w1rel_ref[:]) + eroot_ref[:]
    h = jnp.maximum(h, 0.0)
    h_ref[:] = h
    hpair_ref[0] = h[:, 0:_HC]
    hpair_ref[1] = h[:, _HC:2 * _HC]


def _tc3(agg1p, W1_rel, eroot):
    return pl.pallas_call(
        _tc3_body,
        out_shape=(
            jax.ShapeDtypeStruct((N, EMB1), jnp.float32),
            jax.ShapeDtypeStruct((2, N, _HC), jnp.float32),
        ),
    )(agg1p, W1_rel, eroot)


def _tc5_body(agg2_ref, h_ref, wmu_rel_ref, bmu_ref, wmu_root_ref,
              wstd_rel_ref, bstd_ref, wstd_root_ref, eps_ref, batch_ref,
              wc1_ref, bc1_ref, wc2_ref, bc2_ref, logstd_ref,
              z_ref, zmu_ref, zstd_ref, zn_ref, y_ref, wstd_out_ref):
    h = h_ref[:]

    def mm_t(a, w):
        return jax.lax.dot_general(a, w, (((1,), (1,)), ((), ())),
                                   preferred_element_type=jnp.float32)

    z_mu = jnp.tanh(_mm_halves(agg2_ref, wmu_rel_ref[:]) + bmu_ref[:][None, :]
                    + mm_t(h, wmu_root_ref[:]))
    z_ls = jnp.tanh(_mm_halves(agg2_ref, wstd_rel_ref[:]) + bstd_ref[:][None, :]
                    + mm_t(h, wstd_root_ref[:]))
    z_std = jnp.exp(z_ls)
    z = z_mu + z_std * eps_ref[:]
    zmu_ref[:] = z_mu
    zstd_ref[:] = z_std
    z_ref[:] = z
    zn2 = jnp.sum(z * z, axis=1, keepdims=True)
    rinv = 1.0 / jnp.maximum(jnp.sqrt(zn2), 1e-8)
    zn_ref[:] = z * rinv
    seg = lax.broadcasted_iota(jnp.int32, (G, N), 0)
    mask = (batch_ref[:][None, :] == seg).astype(jnp.float32)
    cnt = jnp.sum(mask, axis=1, keepdims=True)
    pooled = jax.lax.dot_general(mask, z_mu, (((1,), (0,)), ((), ())),
                                 preferred_element_type=jnp.float32)
    pooled = pooled / jnp.maximum(cnt, 1.0)
    y = jnp.maximum(mm_t(pooled, wc1_ref[:]) + bc1_ref[:][None, :], 0.0)
    y = mm_t(y, wc2_ref[:]) + bc2_ref[:][None, :]
    y = y - jnp.max(y, axis=1, keepdims=True)
    ey = jnp.exp(y)
    y_ref[:] = ey / jnp.sum(ey, axis=1, keepdims=True)
    wstd_out_ref[:] = jnp.exp(logstd_ref[:])


def _tc5(agg2, h, Wmu_rel, bmu_rel, Wmu_root, Wstd_rel, bstd_rel, Wstd_root,
         eps, batch, Wc1, bc1, Wc2, bc2, log_std):
    return pl.pallas_call(
        _tc5_body,
        out_shape=(
            jax.ShapeDtypeStruct((N, EMB2), jnp.float32),
            jax.ShapeDtypeStruct((N, EMB2), jnp.float32),
            jax.ShapeDtypeStruct((N, EMB2), jnp.float32),
            jax.ShapeDtypeStruct((N, EMB2), jnp.float32),
            jax.ShapeDtypeStruct((G, 2), jnp.float32),
            jax.ShapeDtypeStruct((1,), jnp.float32),
        ),
    )(agg2, h, Wmu_rel, bmu_rel, Wmu_root, Wstd_rel, bstd_rel, Wstd_root,
      eps, batch, Wc1, bc1, Wc2, bc2, log_std)


def _chunk_sizes(total, cap=128):
    out = []
    while total > 0:
        c = min(cap, total)
        out.append(c)
        total -= c
    return out


@functools.partial(jax.jit, static_argnames=("n_rows", "n_cols"))
def _sc_gather(table, idx, n_rows, n_cols):
    bpw = (-(-n_rows // _NW) + 7) // 8 * 8
    tailw = n_rows - (_NW - 1) * bpw
    assert tailw > 0 and tailw % 8 == 0
    mesh = plsc.VectorSubcoreMesh(core_axis_name="c", subcore_axis_name="s")

    @functools.partial(
        pl.kernel, mesh=mesh,
        out_type=jax.ShapeDtypeStruct((n_rows, n_cols), jnp.float32),
        scratch_types=[
            pltpu.VMEM((bpw,), jnp.int32),
            pltpu.VMEM((bpw, n_cols), jnp.float32),
            pltpu.SemaphoreType.DMA,
        ],
    )
    def k(table_hbm, idx_hbm, out_hbm, idx_v, rows_v, sem):
        wid = lax.axis_index("s") * _NC + lax.axis_index("c")
        base = wid * bpw

        def run(nrows):
            pltpu.sync_copy(idx_hbm.at[pl.ds(base, nrows)],
                            idx_v.at[pl.ds(0, nrows)])
            copies = []
            off = 0
            for cs in _chunk_sizes(nrows):
                copies.append(pltpu.async_copy(
                    table_hbm.at[idx_v.at[pl.ds(off, cs)]],
                    rows_v.at[pl.ds(off, cs)], sem))
                off += cs
            for c in copies:
                c.wait()
            pltpu.sync_copy(rows_v.at[pl.ds(0, nrows)],
                            out_hbm.at[pl.ds(base, nrows)])

        if bpw == tailw:
            run(bpw)
        else:
            @pl.when(wid < _NW - 1)
            def _():
                run(bpw)

            @pl.when(wid == _NW - 1)
            def _():
                run(tailw)

    return k(table, idx)


def _gather_rows(table, idx):
    return table[idx]


_C = 128
_NROW = 10240
_RPS = _NROW // _NS
_HC = 64
_EPS = E // _NS
_NF2 = _EPS // _C
_PC2 = _EPS - _NF2 * _C
_EPW = E // _NW
_NFD = _EPW // _C
_PCD = _EPW - _NFD * _C


@jax.jit
def _sc_segsum(table_pair, edge_index, edge_weight):
    mesh = plsc.VectorSubcoreMesh(core_axis_name="c", subcore_axis_name="s")
    cvecs = _HC // _LANES

    @functools.partial(
        pl.kernel, mesh=mesh,
        out_type=jax.ShapeDtypeStruct((2 * _NROW, _HC), jnp.float32),
        compiler_params=pltpu.CompilerParams(needs_layout_passes=False,
                                             use_tc_tiling_on_sc=False),
        scratch_types=[
            pltpu.VMEM((_EPS,), jnp.int32),
            pltpu.VMEM((_EPS,), jnp.int32),
            pltpu.VMEM((_EPS,), jnp.float32),
            pltpu.VMEM((_C, _HC), jnp.float32),
            pltpu.VMEM((_C, _HC), jnp.float32),
            pltpu.VMEM_SHARED((_NROW, _HC), jnp.float32),
            pltpu.SemaphoreType.DMA,
            pltpu.SemaphoreType.DMA,
        ],
    )
    def k(x_hbm, ei_hbm, ew_hbm, out_hbm,
          src_v, dst_v, ew_v, msg0_v, msg1_v, agg_sh, sem0, sem1):
        cid = lax.axis_index("c")
        sid = lax.axis_index("s")
        zv = jnp.zeros((_LANES,), jnp.float32)

        def zrow(i, c2):
            for c in range(_HC // _LANES):
                msg0_v[i, pl.ds(c * _LANES, _LANES)] = zv
            return c2

        lax.fori_loop(0, _C, zrow, 0)
        for b in range(_RPS // _C):
            pltpu.sync_copy(msg0_v,
                            agg_sh.at[pl.ds(sid * _RPS + b * _C, _C)])
        base = sid * _EPS
        pltpu.sync_copy(ei_hbm.at[0, pl.ds(base, _EPS)], src_v)
        pltpu.sync_copy(ei_hbm.at[1, pl.ds(base, _EPS)], dst_v)
        pltpu.sync_copy(ew_hbm.at[pl.ds(base, _EPS)], ew_v)
        plsc.subcore_barrier()
        xh = x_hbm.at[cid]

        def g_start(j, msg_v, sem, n=_C):
            pltpu.async_copy(xh.at[src_v.at[pl.ds(j * _C, n)]],
                             msg_v.at[pl.ds(0, n)], sem)

        def g_wait(j, msg_v, sem, n=_C):
            pltpu.make_async_copy(xh.at[src_v.at[pl.ds(j * _C, n)]],
                                  msg_v.at[pl.ds(0, n)], sem).wait()

        def scale_and_scatter(j, msg_v, n=_C):
            jbase = jnp.full((_LANES,), j * _C, jnp.int32)

            def srow(i4, c2):
                for u in range(4):
                    i = i4 * 4 + u
                    w = plsc.load_gather(ew_v, [jbase + i])
                    for c in range(cvecs):
                        msg_v[i, pl.ds(c * _LANES, _LANES)] = (
                            msg_v[i, pl.ds(c * _LANES, _LANES)] * w)
                return c2

            lax.fori_loop(0, n // 4, srow, 0)
            pltpu.sync_copy(msg_v.at[pl.ds(0, n)],
                            agg_sh.at[dst_v.at[pl.ds(j * _C, n)]], add=True)

        g_start(0, msg0_v, sem0)

        def pair(jj, carry):
            j = jj * 2
            g_start(j + 1, msg1_v, sem1)
            g_wait(j, msg0_v, sem0)
            scale_and_scatter(j, msg0_v)
            g_start(j + 2, msg0_v, sem0)
            g_wait(j + 1, msg1_v, sem1)
            scale_and_scatter(j + 1, msg1_v)
            return carry

        lax.fori_loop(0, _NF2 // 2 - 1, pair, 0)
        jl = _NF2 - 2
        g_start(jl + 1, msg1_v, sem1)
        g_wait(jl, msg0_v, sem0)
        scale_and_scatter(jl, msg0_v)
        g_start(_NF2, msg0_v, sem0, n=_PC2)
        g_wait(jl + 1, msg1_v, sem1)
        scale_and_scatter(jl + 1, msg1_v)
        g_wait(_NF2, msg0_v, sem0, n=_PC2)
        scale_and_scatter(_NF2, msg0_v, n=_PC2)

        plsc.subcore_barrier()
        pltpu.sync_copy(agg_sh.at[pl.ds(sid * _RPS, _RPS)],
                        out_hbm.at[pl.ds(cid * _NROW + sid * _RPS, _RPS)])

    return k(table_pair, edge_index, edge_weight)


def _segsum(xrows, ew, dst):
    return jax.ops.segment_sum(xrows * ew[:, None], dst, num_segments=N)


@jax.jit
def _sc_edge_dot(zn, edge_index):
    mesh = plsc.VectorSubcoreMesh(core_axis_name="c", subcore_axis_name="s")
    cvecs = EMB2 // _LANES

    @functools.partial(
        pl.kernel, mesh=mesh,
        out_type=jax.ShapeDtypeStruct((E,), jnp.float32),
        compiler_params=pltpu.CompilerParams(needs_layout_passes=False,
                                             use_tc_tiling_on_sc=False),
        scratch_types=[
            pltpu.VMEM((_EPW,), jnp.int32),
            pltpu.VMEM((_EPW,), jnp.int32),
            pltpu.VMEM((_C, EMB2), jnp.float32),
            pltpu.VMEM((_C, EMB2), jnp.float32),
            pltpu.VMEM((_C, EMB2), jnp.float32),
            pltpu.VMEM((_C, EMB2), jnp.float32),
            pltpu.VMEM((_EPW,), jnp.float32),
            pltpu.SemaphoreType.DMA,
            pltpu.SemaphoreType.DMA,
        ],
    )
    def k(zn_hbm, ei_hbm, out_hbm, src_v, dst_v,
          a0_v, b0_v, a1_v, b1_v, o_v, sem0, sem1):
        cid = lax.axis_index("c")
        sid = lax.axis_index("s")
        wid = sid * _NC + cid
        base = wid * _EPW
        pltpu.sync_copy(ei_hbm.at[0, pl.ds(base, _EPW)], src_v)
        pltpu.sync_copy(ei_hbm.at[1, pl.ds(base, _EPW)], dst_v)
        lane = lax.broadcasted_iota(jnp.int32, (_LANES,), 0)
        last = lane == (_LANES - 1)

        def gather_ab(j, a_v, b_v, sem, n=_C):
            pltpu.async_copy(zn_hbm.at[src_v.at[pl.ds(j * _C, n)]],
                             a_v.at[pl.ds(0, n)], sem)
            pltpu.async_copy(zn_hbm.at[dst_v.at[pl.ds(j * _C, n)]],
                             b_v.at[pl.ds(0, n)], sem)

        def wait_ab(j, a_v, b_v, sem, n=_C):
            pltpu.make_async_copy(zn_hbm.at[src_v.at[pl.ds(j * _C, n)]],
                                  a_v.at[pl.ds(0, n)], sem).wait()
            pltpu.make_async_copy(zn_hbm.at[dst_v.at[pl.ds(j * _C, n)]],
                                  b_v.at[pl.ds(0, n)], sem).wait()

        def dots(j, a_v, b_v, n=_C):
            jbase = jnp.full((_LANES,), j * _C, jnp.int32)

            def edge(i8, c2):
                for u in range(8):
                    i = i8 * 8 + u
                    s = (a_v[i, pl.ds(0, _LANES)] * b_v[i, pl.ds(0, _LANES)])
                    for c in range(1, cvecs):
                        s = s + (a_v[i, pl.ds(c * _LANES, _LANES)]
                                 * b_v[i, pl.ds(c * _LANES, _LANES)])
                    cs = plsc.cumsum(s)
                    plsc.store_scatter(o_v, [jbase + i], cs, mask=last)
                return c2

            lax.fori_loop(0, n // 8, edge, 0)

        gather_ab(0, a0_v, b0_v, sem0)

        def pair(jj, carry):
            j = jj * 2
            gather_ab(j + 1, a1_v, b1_v, sem1)
            wait_ab(j, a0_v, b0_v, sem0)
            dots(j, a0_v, b0_v)
            gather_ab(j + 2, a0_v, b0_v, sem0)
            wait_ab(j + 1, a1_v, b1_v, sem1)
            dots(j + 1, a1_v, b1_v)
            return carry

        lax.fori_loop(0, _NFD // 2 - 1, pair, 0)
        jl = _NFD - 2
        gather_ab(jl + 1, a1_v, b1_v, sem1)
        wait_ab(jl, a0_v, b0_v, sem0)
        dots(jl, a0_v, b0_v)
        gather_ab(_NFD, a0_v, b0_v, sem0, n=_PCD)
        wait_ab(jl + 1, a1_v, b1_v, sem1)
        dots(jl + 1, a1_v, b1_v)
        wait_ab(_NFD, a0_v, b0_v, sem0, n=_PCD)
        dots(_NFD, a0_v, b0_v, n=_PCD)
        pltpu.sync_copy(o_v, out_hbm.at[pl.ds(base, _EPW)])

    return k(zn, edge_index)


def kernel(x, edge_index, edge_weight, batch, emb_table, W1_rel, b1_rel,
           W1_root, Wmu_rel, bmu_rel, Wmu_root, Wstd_rel, bstd_rel, Wstd_root,
           Wc1, bc1, Wc2, bc2, log_std, eps):
    e_raw = _sc_gather(emb_table, x, N, HIDDEN)
    e_pair, eroot = _tc1(e_raw, W1_root, b1_rel)

    agg1p = _sc_segsum(e_pair, edge_index, edge_weight)
    h, h_pair = _tc3(agg1p, W1_rel, eroot)

    agg2p = _sc_segsum(h_pair, edge_index, edge_weight)
    z, z_mu, z_std, zn, y, w_std = _tc5(
        agg2p, h, Wmu_rel, bmu_rel, Wmu_root, Wstd_rel, bstd_rel, Wstd_root,
        eps, batch, Wc1, bc1, Wc2, bc2, log_std)

    w_mu = _sc_edge_dot(zn, edge_index)
    return (y, w_mu, w_std, z, z_mu, z_std)
